# baseline (speedup 1.0000x reference)
import functools

import jax
import jax.numpy as jnp
from jax import lax
from jax.experimental import pallas as pl
from jax.experimental.pallas import tpu as pltpu
from jax.experimental.pallas import tpu_sc as plsc

_TOP_K = 8
_NC = 2
_NS = 16
_NW = _NC * _NS
_L = 16


def _gate_matmul_block(x_ref, wt_ref, out_ref):
    out_ref[...] = jnp.dot(x_ref[...], wt_ref[...],
                           preferred_element_type=jnp.float32)


def _gate_logits(x, wt):
    tokens, dim = x.shape
    n_exp = wt.shape[1]
    blk = 512
    return pl.pallas_call(
        _gate_matmul_block,
        grid=(tokens // blk,),
        in_specs=[pl.BlockSpec((blk, dim), lambda i: (i, 0)),
                  pl.BlockSpec((dim, n_exp), lambda i: (0, 0))],
        out_specs=pl.BlockSpec((blk, n_exp), lambda i: (i, 0)),
        out_shape=jax.ShapeDtypeStruct((tokens, n_exp), jnp.float32),
    )(x, wt)


def _make_sc_topk(tokens, n_exp):
    rows = tokens // _NW
    nv = n_exp // _L
    mesh = plsc.VectorSubcoreMesh(core_axis_name="c", subcore_axis_name="s")

    @functools.partial(
        pl.kernel,
        out_type=[jax.ShapeDtypeStruct((tokens * _TOP_K,), jnp.float32),
                  jax.ShapeDtypeStruct((tokens * _TOP_K,), jnp.int32)],
        mesh=mesh,
        scratch_types=[pltpu.VMEM((rows, n_exp), jnp.float32),
                       pltpu.VMEM((rows * _TOP_K + _L,), jnp.float32),
                       pltpu.VMEM((rows * _TOP_K + _L,), jnp.int32)],
        compiler_params=pltpu.CompilerParams(needs_layout_passes=False),
    )
    def sc_topk(logits_hbm, scores_hbm, idx_hbm, lg_v, sc_v, ix_v):
        wid = lax.axis_index("s") * _NC + lax.axis_index("c")
        base = wid * rows
        pltpu.sync_copy(logits_hbm.at[pl.ds(base, rows)], lg_v)
        lane = lax.iota(jnp.int32, _L)
        mask8 = lane < _TOP_K

        def merge(av, ai, bv, bi):
            rbv = lax.rev(bv, (0,))
            rbi = lax.rev(bi, (0,))
            take = av >= rbv
            hv = jnp.where(take, av, rbv)
            hi = jnp.where(take, ai, rbi)
            return plsc.sort_key_val(hv, hi, descending=True)

        def token(r, carry):
            srt = [plsc.sort_key_val(lg_v[r, pl.ds(j * _L, _L)],
                                     lane + j * _L, descending=True)
                   for j in range(nv)]
            while len(srt) > 1:
                srt = [merge(srt[j][0], srt[j][1], srt[j + 1][0], srt[j + 1][1])
                       for j in range(0, len(srt), 2)]
            tv, ti = srt[0]
            mx = jnp.max(tv)
            e = jnp.where(mask8, jnp.exp(tv - mx), 0.0)
            s = e / jnp.sum(e)
            plsc.store_compressed(sc_v.at[pl.ds(r * _TOP_K, _L)], s, mask=mask8)
            plsc.store_compressed(ix_v.at[pl.ds(r * _TOP_K, _L)], ti, mask=mask8)
            return carry

        lax.fori_loop(0, rows, token, 0)
        pltpu.sync_copy(sc_v.at[pl.ds(0, rows * _TOP_K)],
                        scores_hbm.at[pl.ds(base * _TOP_K, rows * _TOP_K)])
        pltpu.sync_copy(ix_v.at[pl.ds(0, rows * _TOP_K)],
                        idx_hbm.at[pl.ds(base * _TOP_K, rows * _TOP_K)])

    return sc_topk


@jax.jit
def kernel(x, gate_w):
    tokens = x.shape[0]
    n_exp = gate_w.shape[0]
    wt = gate_w.T
    logits = _gate_logits(x, wt)
    scores_flat, idx_flat = _make_sc_topk(tokens, n_exp)(logits)
    return (scores_flat.reshape(tokens, _TOP_K),
            idx_flat.reshape(tokens, _TOP_K))


# speedup vs baseline: 1.1827x; 1.1827x over previous
import functools

import jax
import jax.numpy as jnp
from jax import lax
from jax.experimental import pallas as pl
from jax.experimental.pallas import tpu as pltpu
from jax.experimental.pallas import tpu_sc as plsc

_TOP_K = 8
_NC = 2
_NS = 16
_NW = _NC * _NS
_L = 16


def _gate_matmul_block(x_ref, wt_ref, out_ref):
    out_ref[...] = jnp.dot(x_ref[...], wt_ref[...],
                           preferred_element_type=jnp.float32)


def _gate_logits(x, wt, chunk, chunk_tokens):
    dim = x.shape[1]
    n_exp = wt.shape[1]
    blk = 512
    blk0 = chunk * (chunk_tokens // blk)
    return pl.pallas_call(
        _gate_matmul_block,
        grid=(chunk_tokens // blk,),
        in_specs=[pl.BlockSpec((blk, dim), lambda i: (blk0 + i, 0)),
                  pl.BlockSpec((dim, n_exp), lambda i: (0, 0))],
        out_specs=pl.BlockSpec((blk, n_exp), lambda i: (i, 0)),
        out_shape=jax.ShapeDtypeStruct((chunk_tokens, n_exp), jnp.float32),
    )(x, wt)


def _make_sc_topk(tokens, n_exp):
    rows = tokens // _NW
    nv = n_exp // _L
    mesh = plsc.VectorSubcoreMesh(core_axis_name="c", subcore_axis_name="s")

    @functools.partial(
        pl.kernel,
        out_type=[jax.ShapeDtypeStruct((tokens * _TOP_K,), jnp.float32),
                  jax.ShapeDtypeStruct((tokens * _TOP_K,), jnp.int32)],
        mesh=mesh,
        scratch_types=[pltpu.VMEM((rows, n_exp), jnp.float32),
                       pltpu.VMEM((rows * _TOP_K + _L,), jnp.float32),
                       pltpu.VMEM((rows * _TOP_K + _L,), jnp.int32)],
        compiler_params=pltpu.CompilerParams(needs_layout_passes=False),
    )
    def sc_topk(logits_hbm, scores_hbm, idx_hbm, lg_v, sc_v, ix_v):
        wid = lax.axis_index("s") * _NC + lax.axis_index("c")
        base = wid * rows
        pltpu.sync_copy(logits_hbm.at[pl.ds(base, rows)], lg_v)
        lane = lax.iota(jnp.int32, _L)
        mask8 = lane < _TOP_K

        def merge(av, ai, bv, bi):
            rbv = lax.rev(bv, (0,))
            rbi = lax.rev(bi, (0,))
            take = av >= rbv
            hv = jnp.where(take, av, rbv)
            hi = jnp.where(take, ai, rbi)
            return plsc.sort_key_val(hv, hi, descending=True)

        def token(r, carry):
            srt = [plsc.sort_key_val(lg_v[r, pl.ds(j * _L, _L)],
                                     lane + j * _L, descending=True)
                   for j in range(nv)]
            while len(srt) > 1:
                srt = [merge(srt[j][0], srt[j][1], srt[j + 1][0], srt[j + 1][1])
                       for j in range(0, len(srt), 2)]
            tv, ti = srt[0]
            mx = jnp.max(tv)
            e = jnp.where(mask8, jnp.exp(tv - mx), 0.0)
            s = e / jnp.sum(e)
            plsc.store_compressed(sc_v.at[pl.ds(r * _TOP_K, _L)], s, mask=mask8)
            plsc.store_compressed(ix_v.at[pl.ds(r * _TOP_K, _L)], ti, mask=mask8)
            return carry

        lax.fori_loop(0, rows, token, 0)
        pltpu.sync_copy(sc_v.at[pl.ds(0, rows * _TOP_K)],
                        scores_hbm.at[pl.ds(base * _TOP_K, rows * _TOP_K)])
        pltpu.sync_copy(ix_v.at[pl.ds(0, rows * _TOP_K)],
                        idx_hbm.at[pl.ds(base * _TOP_K, rows * _TOP_K)])

    return sc_topk


_NCHUNK = 4


@jax.jit
def kernel(x, gate_w):
    tokens = x.shape[0]
    n_exp = gate_w.shape[0]
    wt = gate_w.T
    ct = tokens // _NCHUNK
    sc_topk = _make_sc_topk(ct, n_exp)
    scores, idxs = [], []
    for c in range(_NCHUNK):
        logits = _gate_logits(x, wt, c, ct)
        s, i = sc_topk(logits)
        scores.append(s.reshape(ct, _TOP_K))
        idxs.append(i.reshape(ct, _TOP_K))
    return jnp.concatenate(scores), jnp.concatenate(idxs)
